# baseline (speedup 1.0000x reference)
import functools

import jax
import jax.numpy as jnp
from jax import lax
from jax.experimental import pallas as pl
from jax.experimental.pallas import tpu as pltpu
from jax.experimental.pallas import tpu_sc as plsc

BATCH = 4096
SEQ = 50
DIM = 128
NC = 2
NS = 16
NW = NC * NS
B_W = BATCH // NW
CH = 64
NCH = SEQ * B_W // CH
NB = 10
NGROUP = NCH // NB


def _emb_body(table_hbm, tok_hbm, out_hbm, idx_v, *rest):
    bufs = rest[:NB]
    gsems = rest[NB:2 * NB]
    wsems = rest[2 * NB:3 * NB]
    wid = lax.axis_index("s") * NC + lax.axis_index("c")
    col_base = wid * B_W
    pltpu.sync_copy(tok_hbm.at[pl.ds(0, SEQ), pl.ds(col_base, B_W)], idx_v)

    def gather_start(b, c):
        s = c // 2
        off = (c % 2) * CH
        pltpu.make_async_copy(table_hbm.at[idx_v.at[s, pl.ds(off, CH)]],
                              bufs[b], gsems[b]).start()

    def gather_wait(b):
        pltpu.make_async_copy(table_hbm.at[pl.ds(0, CH)], bufs[b], gsems[b]).wait()

    def wb_start(b, c):
        s = c // 2
        off = (c % 2) * CH
        pltpu.make_async_copy(bufs[b], out_hbm.at[s, pl.ds(col_base + off, CH)],
                              wsems[b]).start()

    def wb_wait(b):
        pltpu.make_async_copy(bufs[b], out_hbm.at[0, pl.ds(col_base, CH)],
                              wsems[b]).wait()

    for b in range(NB):
        gather_start(b, b)

    def body(g, carry):
        for b in range(NB):
            gather_wait(b)
            wb_start(b, g * NB + b)
        for b in range(NB):
            wb_wait(b)
            gather_start(b, (g + 1) * NB + b)
        return carry

    lax.fori_loop(0, NGROUP - 1, body, 0)

    for b in range(NB):
        gather_wait(b)
        wb_start(b, (NGROUP - 1) * NB + b)
    for b in range(NB):
        wb_wait(b)


@jax.jit
def _embedding_lookup(token_ids, embedding_matrix):
    tok_t = jnp.transpose(token_ids.astype(jnp.int32))
    mesh = plsc.VectorSubcoreMesh(core_axis_name="c", subcore_axis_name="s")
    run = functools.partial(
        pl.kernel,
        mesh=mesh,
        out_type=jax.ShapeDtypeStruct((SEQ, BATCH, DIM), jnp.float32),
        scratch_types=(
            [pltpu.VMEM((SEQ, B_W), jnp.int32)]
            + [pltpu.VMEM((CH, DIM), jnp.float32) for _ in range(NB)]
            + [pltpu.SemaphoreType.DMA for _ in range(2 * NB)]
        ),
    )(_emb_body)
    out = run(embedding_matrix, tok_t)
    return jnp.transpose(out, (1, 0, 2))


def kernel(token_ids, embedding_matrix):
    return _embedding_lookup(token_ids, embedding_matrix)
